# baseline (speedup 1.0000x reference)
import functools

import jax
import jax.numpy as jnp
from jax import lax
from jax.experimental import pallas as pl
from jax.experimental.pallas import tpu as pltpu
from jax.experimental.pallas import tpu_sc as plsc

NC = 2
NS = 16
NW = NC * NS
L = 16
BATCH = 16384
EMB = 16
PACK = 128 // EMB
CH = 128
ROWS = BATCH // CH
NCH = BATCH // (NW * CH)
NV = 1000000
NPK = 131072


def _sc_body(uidx, ridx, uemb, ubias, vemb, vbias, part_out, ubrb_out,
             uidx_v, ridx_v, upk_v, rpk_v, ulan_v, rlan_v, urows_v, vrows_v,
             ub_v, rb_v, ubrb_v, part_v, sem, bsem):
    wid = lax.axis_index("s") * NC + lax.axis_index("c")
    base = wid * NCH
    pltpu.sync_copy(uidx.at[pl.ds(base, NCH)], uidx_v)
    pltpu.sync_copy(ridx.at[pl.ds(base, NCH)], ridx_v)

    for j in range(NCH):
        for g in range(CH // L):
            sl = pl.ds(g * L, L)
            upk_v[j, sl] = uidx_v[j, sl] & (NPK - 1)
            rpk_v[j, sl] = ridx_v[j, sl] & (NPK - 1)
            ulan_v[j, sl] = lax.shift_right_logical(uidx_v[j, sl], 17) * EMB
            rlan_v[j, sl] = lax.shift_right_logical(ridx_v[j, sl], 17) * EMB

    bias_copies = []
    for j in range(NCH):
        bias_copies.append(pltpu.async_copy(
            ubias.at[0].at[uidx_v.at[j]], ub_v.at[j], bsem))
        bias_copies.append(pltpu.async_copy(
            vbias.at[0].at[ridx_v.at[j]], rb_v.at[j], bsem))

    def fire(j, buf):
        return (
            pltpu.async_copy(uemb.at[upk_v.at[j]], urows_v.at[buf], sem),
            pltpu.async_copy(vemb.at[rpk_v.at[j]], vrows_v.at[buf], sem),
        )

    inflight = fire(0, 0)
    acc = jnp.zeros((L,), jnp.float32)
    iota = lax.iota(jnp.int32, L)
    for j in range(NCH):
        buf = j % 2
        for cp in inflight:
            cp.wait()
        if j + 1 < NCH:
            inflight = fire(j + 1, (j + 1) % 2)
        for g in range(CH // L):
            sl = pl.ds(g * L, L)
            rowv = g * L + iota
            ul = ulan_v[j, sl]
            rl = rlan_v[j, sl]

            def feat_body(e, acc):
                uvals = plsc.load_gather(urows_v.at[buf], [rowv, ul + e])
                vvals = plsc.load_gather(vrows_v.at[buf], [rowv, rl + e])
                return acc + uvals * vvals

            acc = lax.fori_loop(0, EMB, feat_body, acc)

    part_v[...] = acc
    pltpu.sync_copy(part_v, part_out.at[wid])

    for cp in bias_copies:
        cp.wait()
    for j in range(NCH):
        for g in range(CH // L):
            sl = pl.ds(g * L, L)
            ubrb_v[j, sl] = ub_v[j, sl] + rb_v[j, sl]
    pltpu.sync_copy(ubrb_v, ubrb_out.at[pl.ds(base, NCH)])


_sc_gather_dot = pl.kernel(
    _sc_body,
    out_type=(
        jax.ShapeDtypeStruct((NW, L), jnp.float32),
        jax.ShapeDtypeStruct((ROWS, CH), jnp.float32),
    ),
    mesh=plsc.VectorSubcoreMesh(core_axis_name="c", subcore_axis_name="s"),
    scratch_types=[
        pltpu.VMEM((NCH, CH), jnp.int32),
        pltpu.VMEM((NCH, CH), jnp.int32),
        pltpu.VMEM((NCH, CH), jnp.int32),
        pltpu.VMEM((NCH, CH), jnp.int32),
        pltpu.VMEM((NCH, CH), jnp.int32),
        pltpu.VMEM((NCH, CH), jnp.int32),
        pltpu.VMEM((2, CH, 128), jnp.float32),
        pltpu.VMEM((2, CH, 128), jnp.float32),
        pltpu.VMEM((NCH, CH), jnp.float32),
        pltpu.VMEM((NCH, CH), jnp.float32),
        pltpu.VMEM((NCH, CH), jnp.float32),
        pltpu.VMEM((L,), jnp.float32),
        pltpu.SemaphoreType.DMA,
        pltpu.SemaphoreType.DMA,
    ],
    compiler_params=pltpu.CompilerParams(
        use_tc_tiling_on_sc=False, needs_layout_passes=False
    ),
)


def _combine_body(part_ref, ubrb_ref, out_ref):
    s = jnp.sum(part_ref[...])
    out_ref[...] = jax.nn.sigmoid(s + ubrb_ref[...])


_combine = pl.pallas_call(
    _combine_body,
    out_shape=jax.ShapeDtypeStruct((ROWS, CH), jnp.float32),
)

RB = 4096
RJ = NPK // RB
_IN_BLOCKS = -(-NV // RB)


def _make_in_map(w):
    def in_map(j):
        return (0, jnp.minimum(w * RJ + j, _IN_BLOCKS - 1))
    return in_map


def _repack_body(*refs):
    in_refs, out_refs, scratch = refs[: 2 * PACK], refs[2 * PACK: 2 * PACK + 2], refs[-1]
    for t in range(2):
        for w in range(PACK):
            scratch[pl.ds(w * EMB, EMB), :] = in_refs[t * PACK + w][...]
        out_refs[t][...] = scratch[...].T


_repack = pl.pallas_call(
    _repack_body,
    grid=(RJ,),
    in_specs=[
        pl.BlockSpec((EMB, RB), _make_in_map(w))
        for _ in range(2)
        for w in range(PACK)
    ],
    out_specs=[
        pl.BlockSpec((RB, 128), lambda j: (j, 0)),
        pl.BlockSpec((RB, 128), lambda j: (j, 0)),
    ],
    out_shape=[
        jax.ShapeDtypeStruct((NPK, 128), jnp.float32),
        jax.ShapeDtypeStruct((NPK, 128), jnp.float32),
    ],
    scratch_shapes=[pltpu.VMEM((128, RB), jnp.float32)],
)


@jax.jit
def kernel(inputs, user_embedding, user_bias, movie_embedding, movie_bias):
    uidx = inputs[:, 0].astype(jnp.int32).reshape(ROWS, CH)
    ridx = inputs[:, 1].astype(jnp.int32).reshape(ROWS, CH)
    ut = user_embedding.T
    vt = movie_embedding.T
    upk, vpk = _repack(*([ut] * PACK), *([vt] * PACK))
    part, ubrb = _sc_gather_dot(
        uidx, ridx,
        upk, user_bias.T,
        vpk, movie_bias.T,
    )
    return _combine(part, ubrb).reshape(BATCH, 1)


# speedup vs baseline: 1.4815x; 1.4815x over previous
import functools

import jax
import jax.numpy as jnp
from jax import lax
from jax.experimental import pallas as pl
from jax.experimental.pallas import tpu as pltpu
from jax.experimental.pallas import tpu_sc as plsc

NC = 2
NS = 16
NW = NC * NS
L = 16
BATCH = 16384
EMB = 16
PACK = 128 // EMB
CH = 128
ROWS = BATCH // CH
NCH = BATCH // (NW * CH)
NV = 1000000
NPK = 131072


def _sc_body(uidx, ridx, uemb, ubias, vemb, vbias, part_out, ubrb_out,
             uidx_v, ridx_v, upk_v, rpk_v, ulan_v, rlan_v, urows_v, vrows_v,
             ubrb_v, part_v, sem, bsem):
    wid = lax.axis_index("s") * NC + lax.axis_index("c")
    base = wid * NCH
    pltpu.sync_copy(uidx.at[pl.ds(base, NCH)], uidx_v)
    pltpu.sync_copy(ridx.at[pl.ds(base, NCH)], ridx_v)

    for j in range(NCH):
        for g in range(CH // L):
            sl = pl.ds(g * L, L)
            upk_v[j, sl] = uidx_v[j, sl] & (NPK - 1)
            rpk_v[j, sl] = ridx_v[j, sl] & (NPK - 1)
            ulan_v[j, sl] = lax.shift_right_logical(uidx_v[j, sl], 17) * EMB
            rlan_v[j, sl] = lax.shift_right_logical(ridx_v[j, sl], 17) * EMB

    def fire(j, buf):
        return (
            pltpu.async_copy(uemb.at[upk_v.at[j]], urows_v.at[buf], sem),
            pltpu.async_copy(vemb.at[rpk_v.at[j]], vrows_v.at[buf], sem),
        )

    inflight = fire(0, 0)
    acc = jnp.zeros((L,), jnp.float32)
    iota = lax.iota(jnp.int32, L)
    for j in range(NCH):
        buf = j % 2
        for cp in inflight:
            cp.wait()
        if j + 1 < NCH:
            inflight = fire(j + 1, (j + 1) % 2)
        for g in range(CH // L):
            sl = pl.ds(g * L, L)
            rowv = g * L + iota
            ul = ulan_v[j, sl]
            rl = rlan_v[j, sl]

            def feat_body(e, acc):
                uvals = plsc.load_gather(urows_v.at[buf], [rowv, ul + e])
                vvals = plsc.load_gather(vrows_v.at[buf], [rowv, rl + e])
                return acc + uvals * vvals

            acc = lax.fori_loop(0, EMB, feat_body, acc)

    part_v[...] = acc
    pltpu.sync_copy(part_v, part_out.at[wid])

    for j in range(NCH):
        for g in range(CH // L):
            sl = pl.ds(g * L, L)
            upk_v[j, sl] = lax.shift_right_logical(uidx_v[j, sl], 7)
            rpk_v[j, sl] = lax.shift_right_logical(ridx_v[j, sl], 7)

    def fireb(j, buf):
        return (
            pltpu.async_copy(ubias.at[upk_v.at[j]], urows_v.at[buf], bsem),
            pltpu.async_copy(vbias.at[rpk_v.at[j]], vrows_v.at[buf], bsem),
        )

    inflight = fireb(0, 0)
    for j in range(NCH):
        buf = j % 2
        for cp in inflight:
            cp.wait()
        if j + 1 < NCH:
            inflight = fireb(j + 1, (j + 1) % 2)
        for g in range(CH // L):
            sl = pl.ds(g * L, L)
            rowv = g * L + iota
            ub16 = plsc.load_gather(urows_v.at[buf], [rowv, uidx_v[j, sl] & 127])
            rb16 = plsc.load_gather(vrows_v.at[buf], [rowv, ridx_v[j, sl] & 127])
            ubrb_v[j, sl] = ub16 + rb16
    pltpu.sync_copy(ubrb_v, ubrb_out.at[pl.ds(base, NCH)])


_sc_gather_dot = pl.kernel(
    _sc_body,
    out_type=(
        jax.ShapeDtypeStruct((NW, L), jnp.float32),
        jax.ShapeDtypeStruct((ROWS, CH), jnp.float32),
    ),
    mesh=plsc.VectorSubcoreMesh(core_axis_name="c", subcore_axis_name="s"),
    scratch_types=[
        pltpu.VMEM((NCH, CH), jnp.int32),
        pltpu.VMEM((NCH, CH), jnp.int32),
        pltpu.VMEM((NCH, CH), jnp.int32),
        pltpu.VMEM((NCH, CH), jnp.int32),
        pltpu.VMEM((NCH, CH), jnp.int32),
        pltpu.VMEM((NCH, CH), jnp.int32),
        pltpu.VMEM((2, CH, 128), jnp.float32),
        pltpu.VMEM((2, CH, 128), jnp.float32),
        pltpu.VMEM((NCH, CH), jnp.float32),
        pltpu.VMEM((L,), jnp.float32),
        pltpu.SemaphoreType.DMA,
        pltpu.SemaphoreType.DMA,
    ],
    compiler_params=pltpu.CompilerParams(
        use_tc_tiling_on_sc=False, needs_layout_passes=False
    ),
)


def _combine_body(part_ref, ubrb_ref, out_ref):
    s = jnp.sum(part_ref[...])
    out_ref[...] = jax.nn.sigmoid(s + ubrb_ref[...])


_combine = pl.pallas_call(
    _combine_body,
    out_shape=jax.ShapeDtypeStruct((ROWS, CH), jnp.float32),
)

RB = 4096
RJ = NPK // RB
_IN_BLOCKS = -(-NV // RB)
BCH = 32768
NBL = RJ * (BCH // 128)
_BIAS_BLOCKS = -(-NV // BCH)


def _bias_in_map(j):
    return (0, jnp.minimum(j, _BIAS_BLOCKS - 1))


def _make_in_map(w):
    def in_map(j):
        return (0, jnp.minimum(w * RJ + j, _IN_BLOCKS - 1))
    return in_map


def _repack_body(*refs):
    in_refs = refs[: 2 * PACK]
    ub_ref, vb_ref = refs[2 * PACK: 2 * PACK + 2]
    out_refs = refs[2 * PACK + 2: 2 * PACK + 4]
    ubp_ref, vbp_ref = refs[2 * PACK + 4: 2 * PACK + 6]
    scratch = refs[-1]
    for t in range(2):
        for w in range(PACK):
            scratch[pl.ds(w * EMB, EMB), :] = in_refs[t * PACK + w][...]
        out_refs[t][...] = scratch[...].T
    ubp_ref[...] = ub_ref[...].reshape(BCH // 128, 128)
    vbp_ref[...] = vb_ref[...].reshape(BCH // 128, 128)


_repack = pl.pallas_call(
    _repack_body,
    grid=(RJ,),
    in_specs=[
        pl.BlockSpec((EMB, RB), _make_in_map(w))
        for _ in range(2)
        for w in range(PACK)
    ] + [
        pl.BlockSpec((1, BCH), _bias_in_map),
        pl.BlockSpec((1, BCH), _bias_in_map),
    ],
    out_specs=[
        pl.BlockSpec((RB, 128), lambda j: (j, 0)),
        pl.BlockSpec((RB, 128), lambda j: (j, 0)),
        pl.BlockSpec((BCH // 128, 128), lambda j: (j, 0)),
        pl.BlockSpec((BCH // 128, 128), lambda j: (j, 0)),
    ],
    out_shape=[
        jax.ShapeDtypeStruct((NPK, 128), jnp.float32),
        jax.ShapeDtypeStruct((NPK, 128), jnp.float32),
        jax.ShapeDtypeStruct((NBL, 128), jnp.float32),
        jax.ShapeDtypeStruct((NBL, 128), jnp.float32),
    ],
    scratch_shapes=[pltpu.VMEM((128, RB), jnp.float32)],
)


@jax.jit
def kernel(inputs, user_embedding, user_bias, movie_embedding, movie_bias):
    uidx = inputs[:, 0].astype(jnp.int32).reshape(ROWS, CH)
    ridx = inputs[:, 1].astype(jnp.int32).reshape(ROWS, CH)
    ut = user_embedding.T
    vt = movie_embedding.T
    upk, vpk, ubp, vbp = _repack(
        *([ut] * PACK), *([vt] * PACK), user_bias.T, movie_bias.T)
    part, ubrb = _sc_gather_dot(
        uidx, ridx,
        upk, ubp,
        vpk, vbp,
    )
    return _combine(part, ubrb).reshape(BATCH, 1)


# speedup vs baseline: 1.6728x; 1.1291x over previous
import functools

import jax
import jax.numpy as jnp
from jax import lax
from jax.experimental import pallas as pl
from jax.experimental.pallas import tpu as pltpu
from jax.experimental.pallas import tpu_sc as plsc

NC = 2
NS = 16
NW = NC * NS
L = 16
BATCH = 16384
EMB = 16
PACK = 128 // EMB
CH = 128
ROWS = BATCH // CH
NCH = BATCH // (NW * CH)
NV = 1000000
NPK = 131072


def _sc_body(uidx, ridx, uemb, ubias, vemb, vbias, part_out, ubrb_out,
             uidx_v, ridx_v, upk_v, rpk_v, urows_v, vrows_v,
             ub_v, rb_v, ubrb_v, part_v, sem, bsem):
    wid = lax.axis_index("s") * NC + lax.axis_index("c")
    base = wid * NCH
    pltpu.sync_copy(uidx.at[pl.ds(base, NCH)], uidx_v)
    pltpu.sync_copy(ridx.at[pl.ds(base, NCH)], ridx_v)

    for j in range(NCH):
        for g in range(CH // L):
            sl = pl.ds(g * L, L)
            upk_v[j, sl] = ((uidx_v[j, sl] & (NPK - 1)) << 3) | lax.shift_right_logical(uidx_v[j, sl], 17)
            rpk_v[j, sl] = ((ridx_v[j, sl] & (NPK - 1)) << 3) | lax.shift_right_logical(ridx_v[j, sl], 17)

    bias_copies = []
    for j in range(NCH):
        bias_copies.append(pltpu.async_copy(ubias.at[uidx_v.at[j]], ub_v.at[j], bsem))
        bias_copies.append(pltpu.async_copy(vbias.at[ridx_v.at[j]], rb_v.at[j], bsem))

    def fire(j, buf):
        return (
            pltpu.async_copy(uemb.at[upk_v.at[j]], urows_v.at[buf], sem),
            pltpu.async_copy(vemb.at[rpk_v.at[j]], vrows_v.at[buf], sem),
        )

    inflight = fire(0, 0)
    acc = jnp.zeros((L,), jnp.float32)
    for j in range(NCH):
        buf = j % 2
        for cp in inflight:
            cp.wait()
        if j + 1 < NCH:
            inflight = fire(j + 1, (j + 1) % 2)

        def row_body(r, a):
            return a + urows_v[buf, r, :] * vrows_v[buf, r, :]

        acc = lax.fori_loop(0, CH, row_body, acc)

    part_v[...] = acc
    pltpu.sync_copy(part_v, part_out.at[wid])

    for cp in bias_copies:
        cp.wait()
    for j in range(NCH):
        for g in range(CH // L):
            sl = pl.ds(g * L, L)
            ubrb_v[j, sl] = ub_v[j, sl] + rb_v[j, sl]
    pltpu.sync_copy(ubrb_v, ubrb_out.at[pl.ds(base, NCH)])


_sc_gather_dot = pl.kernel(
    _sc_body,
    out_type=(
        jax.ShapeDtypeStruct((NW, L), jnp.float32),
        jax.ShapeDtypeStruct((ROWS, CH), jnp.float32),
    ),
    mesh=plsc.VectorSubcoreMesh(core_axis_name="c", subcore_axis_name="s"),
    scratch_types=[
        pltpu.VMEM((NCH, CH), jnp.int32),
        pltpu.VMEM((NCH, CH), jnp.int32),
        pltpu.VMEM((NCH, CH), jnp.int32),
        pltpu.VMEM((NCH, CH), jnp.int32),
        pltpu.VMEM((2, CH, EMB), jnp.float32),
        pltpu.VMEM((2, CH, EMB), jnp.float32),
        pltpu.VMEM((NCH, CH), jnp.float32),
        pltpu.VMEM((NCH, CH), jnp.float32),
        pltpu.VMEM((NCH, CH), jnp.float32),
        pltpu.VMEM((L,), jnp.float32),
        pltpu.SemaphoreType.DMA,
        pltpu.SemaphoreType.DMA,
    ],
    compiler_params=pltpu.CompilerParams(
        use_tc_tiling_on_sc=False, needs_layout_passes=False
    ),
)


def _combine_body(part_ref, ubrb_ref, out_ref):
    s = jnp.sum(part_ref[...])
    out_ref[...] = jax.nn.sigmoid(s + ubrb_ref[...])


_combine = pl.pallas_call(
    _combine_body,
    out_shape=jax.ShapeDtypeStruct((ROWS, CH), jnp.float32),
)

RB = 4096
RJ = NPK // RB
_IN_BLOCKS = -(-NV // RB)
BCH = 32768
NBL = RJ * (BCH // 128)
_BIAS_BLOCKS = -(-NV // BCH)


def _bias_in_map(j):
    return (0, jnp.minimum(j, _BIAS_BLOCKS - 1))


def _make_in_map(w):
    def in_map(j):
        return (0, jnp.minimum(w * RJ + j, _IN_BLOCKS - 1))
    return in_map


def _repack_body(*refs):
    in_refs = refs[: 2 * PACK]
    ub_ref, vb_ref = refs[2 * PACK: 2 * PACK + 2]
    out_refs = refs[2 * PACK + 2: 2 * PACK + 4]
    ubp_ref, vbp_ref = refs[2 * PACK + 4: 2 * PACK + 6]
    scratch = refs[-1]
    for t in range(2):
        for w in range(PACK):
            scratch[pl.ds(w * EMB, EMB), :] = in_refs[t * PACK + w][...]
        out_refs[t][...] = scratch[...].T
    ubp_ref[...] = ub_ref[...].reshape(BCH // 128, 128)
    vbp_ref[...] = vb_ref[...].reshape(BCH // 128, 128)


_repack = pl.pallas_call(
    _repack_body,
    grid=(RJ,),
    in_specs=[
        pl.BlockSpec((EMB, RB), _make_in_map(w))
        for _ in range(2)
        for w in range(PACK)
    ] + [
        pl.BlockSpec((1, BCH), _bias_in_map),
        pl.BlockSpec((1, BCH), _bias_in_map),
    ],
    out_specs=[
        pl.BlockSpec((RB, 128), lambda j: (j, 0)),
        pl.BlockSpec((RB, 128), lambda j: (j, 0)),
        pl.BlockSpec((BCH // 128, 128), lambda j: (j, 0)),
        pl.BlockSpec((BCH // 128, 128), lambda j: (j, 0)),
    ],
    out_shape=[
        jax.ShapeDtypeStruct((NPK, 128), jnp.float32),
        jax.ShapeDtypeStruct((NPK, 128), jnp.float32),
        jax.ShapeDtypeStruct((NBL, 128), jnp.float32),
        jax.ShapeDtypeStruct((NBL, 128), jnp.float32),
    ],
    scratch_shapes=[pltpu.VMEM((128, RB), jnp.float32)],
)


@jax.jit
def kernel(inputs, user_embedding, user_bias, movie_embedding, movie_bias):
    uidx = inputs[:, 0].astype(jnp.int32).reshape(ROWS, CH)
    ridx = inputs[:, 1].astype(jnp.int32).reshape(ROWS, CH)
    ut = user_embedding.T
    vt = movie_embedding.T
    upk, vpk, ubp, vbp = _repack(
        *([ut] * PACK), *([vt] * PACK), user_bias.T, movie_bias.T)
    part, ubrb = _sc_gather_dot(
        uidx, ridx,
        upk.reshape(-1, EMB), ubp.reshape(-1),
        vpk.reshape(-1, EMB), vbp.reshape(-1),
    )
    return _combine(part, ubrb).reshape(BATCH, 1)
